# baseline (speedup 1.0000x reference)
import functools

import jax
import jax.numpy as jnp
from jax import lax
from jax.experimental import pallas as pl
from jax.experimental.pallas import tpu as pltpu
from jax.experimental.pallas import tpu_sc as plsc

N_NODES = 10000
N_PAD = 10240
CH_HALF = 64
CHUNK = 128
N_SUBCORES = 16
N_CORES = 2
NBUF = 4
TILE_ROWS = N_PAD // N_SUBCORES


def _sc_body(nch,
             xr, idxall, ones_h, w, out,
             accb, accd, idxb, xi, xj, onesb, degb, wbuf,
             gsem0, gsem1, gsem2, gsem3, ssem0, ssem1, ssem2, ssem3,
             isem0, isem1, isem2, isem3, isem4, isem5, isem6, isem7):
    c = lax.axis_index("c")
    s = lax.axis_index("s")
    gsem = (gsem0, gsem1, gsem2, gsem3)
    ssem = (ssem0, ssem1, ssem2, ssem3)
    isem = (isem0, isem1, isem2, isem3, isem4, isem5, isem6, isem7)
    rows0 = s * TILE_ROWS

    def _idx_load(k, slot):
        pltpu.async_copy(idxall.at[c, s * nch + k], idxb.at[slot],
                         isem[slot])

    _idx_load(0, 0)
    _idx_load(1, 1)
    _idx_load(2, 2)

    def _zrow(r, carry):
        for kk in range(4):
            xi[0, r, pl.ds(16 * kk, 16)] = jnp.zeros((16,), jnp.float32)
        return carry
    lax.fori_loop(0, CHUNK, _zrow, 0)
    for kk in range(CHUNK // 16):
        degb[pl.ds(16 * kk, 16)] = jnp.zeros((16,), jnp.float32)
    for r5 in range(TILE_ROWS // CHUNK):
        pltpu.sync_copy(xi.at[0], accb.at[pl.ds(rows0 + CHUNK * r5, CHUNK)])
        pltpu.sync_copy(degb, accd.at[pl.ds(rows0 + CHUNK * r5, CHUNK)])
    pltpu.sync_copy(ones_h, onesb)
    plsc.subcore_barrier()

    base_g = s * nch

    def _drain_big(sem):
        pltpu.make_async_copy(xr.at[pl.ds(0, CHUNK)], xj.at[0], sem).wait()

    def _drain_small(sem):
        pltpu.make_async_copy(ones_h, degb, sem).wait()

    NIB = 2 * NBUF

    def _idx_wait(slot):
        pltpu.make_async_copy(idxall.at[c, base_g], idxb.at[slot],
                              isem[slot]).wait()

    def _gathers(k, b, slot):
        pltpu.async_copy(xr.at[idxb.at[slot, 0]], xi.at[b], gsem[b])
        pltpu.async_copy(xr.at[idxb.at[slot, 1]], xj.at[b], gsem[b])

    _idx_wait(0)
    _gathers(0, 0, 0)

    def _round(t, carry):
        for p in range(NIB):
            k = NIB * t + p
            b = p % NBUF
            q = (p + 1) % NBUF

            @pl.when(k >= NBUF - 1)
            def _():
                _drain_big(ssem[q])
                _drain_big(ssem[q])
                _drain_small(ssem[q])
                _drain_small(ssem[q])

            @pl.when(k + 3 < nch)
            def _():
                _idx_load(k + 3, (p + 3) % NIB)

            @pl.when(k + 1 < nch)
            def _():
                _idx_wait((p + 1) % NIB)
                _gathers(k + 1, q, (p + 1) % NIB)

            _drain_big(gsem[b])
            _drain_big(gsem[b])

            pltpu.async_copy(xj.at[b], accb.at[idxb.at[p, 2]], ssem[b],
                             add=True)
            pltpu.async_copy(xi.at[b], accb.at[idxb.at[p, 3]], ssem[b],
                             add=True)
            pltpu.async_copy(onesb, accd.at[idxb.at[p, 2]], ssem[b],
                             add=True)
            pltpu.async_copy(onesb, accd.at[idxb.at[p, 3]], ssem[b],
                             add=True)
        return carry

    lax.fori_loop(0, nch // NIB, _round, 0)
    for k in range(nch - (NBUF - 1), nch):
        _drain_big(ssem[k % NBUF])
        _drain_big(ssem[k % NBUF])
        _drain_small(ssem[k % NBUF])
        _drain_small(ssem[k % NBUF])
    plsc.subcore_barrier()

    pltpu.sync_copy(w, wbuf)
    wv = wbuf[...]
    w2 = wv * wv
    for r5 in range(TILE_ROWS // CHUNK):
        rows = rows0 + CHUNK * r5
        pltpu.sync_copy(xr.at[pl.ds(c * N_PAD + rows, CHUNK)], xi.at[0])
        pltpu.sync_copy(accb.at[pl.ds(rows, CHUNK)], xj.at[0])
        pltpu.sync_copy(accd.at[pl.ds(rows, CHUNK)], degb)

        def _sblk(b, carry):
            d16 = degb[pl.ds(16 * b, 16)]
            for l in range(16):
                r = 16 * b + l
                dl = d16[l]
                for kk in range(4):
                    sl = pl.ds(16 * kk, 16)
                    xi[1, r, sl] = (xi[0, r, sl] * dl - xj[0, r, sl]) * w2
            return carry
        lax.fori_loop(0, CHUNK // 16, _sblk, 0)
        pltpu.sync_copy(xi.at[1], out.at[pl.ds(c * N_PAD + rows, CHUNK)])


@functools.partial(jax.jit, static_argnames=("nch",))
def _run(xr, idxall, ones_h, w16, nch):
    mesh = plsc.VectorSubcoreMesh(core_axis_name="c", subcore_axis_name="s")
    body = functools.partial(_sc_body, nch)
    return pl.kernel(
        body,
        out_type=jax.ShapeDtypeStruct((N_CORES * N_PAD, CH_HALF),
                                      jnp.float32),
        mesh=mesh,
        compiler_params=pltpu.CompilerParams(use_tc_tiling_on_sc=False),
        scratch_types=[
            pltpu.VMEM_SHARED((N_PAD, CH_HALF), jnp.float32),
            pltpu.VMEM_SHARED((N_PAD,), jnp.float32),
            pltpu.VMEM((2 * NBUF, 4, CHUNK), jnp.int32),
            pltpu.VMEM((NBUF, CHUNK, CH_HALF), jnp.float32),
            pltpu.VMEM((NBUF, CHUNK, CH_HALF), jnp.float32),
            pltpu.VMEM((CHUNK,), jnp.float32),
            pltpu.VMEM((CHUNK,), jnp.float32),
            pltpu.VMEM((16,), jnp.float32),
        ] + [pltpu.SemaphoreType.DMA] * 16,
    )(xr, idxall, ones_h, w16)


def kernel(x, iInd, jInd, W):
    n = x.shape[2]
    e = iInd.shape[0]
    nch = -(-e // (N_SUBCORES * CHUNK))
    nch = -(-nch // (2 * NBUF)) * (2 * NBUF)
    e_pad = nch * N_SUBCORES * CHUNK

    xt = x[0].reshape(N_CORES, CH_HALF, n).transpose(0, 2, 1)
    xr = jnp.zeros((N_CORES, N_PAD, CH_HALF), x.dtype).at[:, :n, :].set(xt)
    xr = xr.reshape(N_CORES * N_PAD, CH_HALF)

    spread = (jnp.arange(e_pad, dtype=jnp.int32) * 37) % n
    ii = spread.at[:e].set(iInd.astype(jnp.int32))
    jj = spread.at[:e].set(jInd.astype(jnp.int32))
    gi = jnp.stack([ii, ii + N_PAD])
    gj = jnp.stack([jj, jj + N_PAD])
    si = jnp.broadcast_to(ii, (N_CORES, e_pad))
    sj = jnp.broadcast_to(jj, (N_CORES, e_pad))
    idxall = jnp.stack([gi, gj, si, sj], axis=1)
    idxall = idxall.reshape(N_CORES, 4, e_pad // CHUNK, CHUNK)
    idxall = idxall.transpose(0, 2, 1, 3)

    ones_h = jnp.ones((CHUNK,), jnp.float32)
    w16 = jnp.broadcast_to(W.astype(jnp.float32).reshape(()), (16,))

    o = _run(xr, idxall, ones_h, w16, nch)
    out = jnp.concatenate([o[:n], o[N_PAD:N_PAD + n]], axis=1).T[None]
    return out


# speedup vs baseline: 1.0715x; 1.0715x over previous
import functools

import jax
import jax.numpy as jnp
from jax import lax
from jax.experimental import pallas as pl
from jax.experimental.pallas import tpu as pltpu
from jax.experimental.pallas import tpu_sc as plsc

N_NODES = 10000
N_PAD = 10240
CH_HALF = 64
CHUNK = 128
N_SUBCORES = 16
N_CORES = 2
NBUF = 4
TILE_ROWS = N_PAD // N_SUBCORES


def _sc_body(nch,
             xr, idxall, ones_h, w, out,
             accb, accd, idxb, xi, xj, onesb, degb, wbuf,
             gsem0, gsem1, gsem2, gsem3, ssem0, ssem1, ssem2, ssem3,
             isem0, isem1, isem2, isem3, isem4, isem5, isem6, isem7):
    c = lax.axis_index("c")
    s = lax.axis_index("s")
    gsem = (gsem0, gsem1, gsem2, gsem3)
    ssem = (ssem0, ssem1, ssem2, ssem3)
    isem = (isem0, isem1, isem2, isem3, isem4, isem5, isem6, isem7)
    rows0 = s * TILE_ROWS

    def _zrow(r, carry):
        for kk in range(4):
            xi[0, r, pl.ds(16 * kk, 16)] = jnp.zeros((16,), jnp.float32)
        return carry
    lax.fori_loop(0, CHUNK, _zrow, 0)
    for kk in range(CHUNK // 16):
        degb[pl.ds(16 * kk, 16)] = jnp.zeros((16,), jnp.float32)
    for r5 in range(TILE_ROWS // CHUNK):
        pltpu.sync_copy(xi.at[0], accb.at[pl.ds(rows0 + CHUNK * r5, CHUNK)])
        pltpu.sync_copy(degb, accd.at[pl.ds(rows0 + CHUNK * r5, CHUNK)])
    pltpu.sync_copy(ones_h, onesb)
    plsc.subcore_barrier()

    base_g = s * nch

    def _drain_big(sem):
        pltpu.make_async_copy(xr.at[pl.ds(0, CHUNK)], xj.at[0], sem).wait()

    def _drain_small(sem):
        pltpu.make_async_copy(ones_h, degb, sem).wait()

    NIB = 2 * NBUF

    def _idx_load(k, slot):
        pltpu.async_copy(idxall.at[c, base_g + k], idxb.at[slot], isem[slot])

    def _idx_wait(slot):
        pltpu.make_async_copy(idxall.at[c, base_g], idxb.at[slot],
                              isem[slot]).wait()

    def _gathers(k, b, slot):
        pltpu.async_copy(xr.at[idxb.at[slot, 0]], xi.at[b], gsem[b])
        pltpu.async_copy(xr.at[idxb.at[slot, 1]], xj.at[b], gsem[b])

    _idx_load(0, 0)
    _idx_load(1, 1)
    _idx_wait(0)
    _gathers(0, 0, 0)

    def _round(t, carry):
        for p in range(NIB):
            k = NIB * t + p
            b = p % NBUF
            q = (p + 1) % NBUF

            @pl.when(k >= NBUF - 1)
            def _():
                _drain_big(ssem[q])
                _drain_big(ssem[q])
                _drain_small(ssem[q])
                _drain_small(ssem[q])

            @pl.when(k + 2 < nch)
            def _():
                _idx_load(k + 2, (p + 2) % NIB)

            @pl.when(k + 1 < nch)
            def _():
                _idx_wait((p + 1) % NIB)
                _gathers(k + 1, q, (p + 1) % NIB)

            _drain_big(gsem[b])
            _drain_big(gsem[b])

            pltpu.async_copy(xj.at[b], accb.at[idxb.at[p, 2]], ssem[b],
                             add=True)
            pltpu.async_copy(xi.at[b], accb.at[idxb.at[p, 3]], ssem[b],
                             add=True)
            pltpu.async_copy(onesb, accd.at[idxb.at[p, 2]], ssem[b],
                             add=True)
            pltpu.async_copy(onesb, accd.at[idxb.at[p, 3]], ssem[b],
                             add=True)
        return carry

    lax.fori_loop(0, nch // NIB, _round, 0)
    for k in range(nch - (NBUF - 1), nch):
        _drain_big(ssem[k % NBUF])
        _drain_big(ssem[k % NBUF])
        _drain_small(ssem[k % NBUF])
        _drain_small(ssem[k % NBUF])
    plsc.subcore_barrier()

    pltpu.sync_copy(w, wbuf)
    wv = wbuf[...]
    w2 = wv * wv
    for r5 in range(TILE_ROWS // CHUNK):
        rows = rows0 + CHUNK * r5
        pltpu.sync_copy(xr.at[pl.ds(c * N_PAD + rows, CHUNK)], xi.at[0])
        pltpu.sync_copy(accb.at[pl.ds(rows, CHUNK)], xj.at[0])
        pltpu.sync_copy(accd.at[pl.ds(rows, CHUNK)], degb)

        def _sblk(b, carry):
            d16 = degb[pl.ds(16 * b, 16)]
            for l in range(16):
                r = 16 * b + l
                dl = d16[l]
                for kk in range(4):
                    sl = pl.ds(16 * kk, 16)
                    xi[1, r, sl] = (xi[0, r, sl] * dl - xj[0, r, sl]) * w2
            return carry
        lax.fori_loop(0, CHUNK // 16, _sblk, 0)
        pltpu.sync_copy(xi.at[1], out.at[pl.ds(c * N_PAD + rows, CHUNK)])


@functools.partial(jax.jit, static_argnames=("nch",))
def _run(xr, idxall, ones_h, w16, nch):
    mesh = plsc.VectorSubcoreMesh(core_axis_name="c", subcore_axis_name="s")
    body = functools.partial(_sc_body, nch)
    return pl.kernel(
        body,
        out_type=jax.ShapeDtypeStruct((N_CORES * N_PAD, CH_HALF),
                                      jnp.float32),
        mesh=mesh,
        compiler_params=pltpu.CompilerParams(use_tc_tiling_on_sc=False),
        scratch_types=[
            pltpu.VMEM_SHARED((N_PAD, CH_HALF), jnp.float32),
            pltpu.VMEM_SHARED((N_PAD,), jnp.float32),
            pltpu.VMEM((2 * NBUF, 4, CHUNK), jnp.int32),
            pltpu.VMEM((NBUF, CHUNK, CH_HALF), jnp.float32),
            pltpu.VMEM((NBUF, CHUNK, CH_HALF), jnp.float32),
            pltpu.VMEM((CHUNK,), jnp.float32),
            pltpu.VMEM((CHUNK,), jnp.float32),
            pltpu.VMEM((16,), jnp.float32),
        ] + [pltpu.SemaphoreType.DMA] * 16,
    )(xr, idxall, ones_h, w16)


def kernel(x, iInd, jInd, W):
    n = x.shape[2]
    e = iInd.shape[0]
    nch = -(-e // (N_SUBCORES * CHUNK))
    nch = -(-nch // (2 * NBUF)) * (2 * NBUF)
    e_pad = nch * N_SUBCORES * CHUNK

    xt = x[0].reshape(N_CORES, CH_HALF, n).transpose(0, 2, 1)
    xr = jnp.zeros((N_CORES, N_PAD, CH_HALF), x.dtype).at[:, :n, :].set(xt)
    xr = xr.reshape(N_CORES * N_PAD, CH_HALF)

    spread = (jnp.arange(e_pad, dtype=jnp.int32) * 37) % n
    ii = spread.at[:e].set(iInd.astype(jnp.int32))
    jj = spread.at[:e].set(jInd.astype(jnp.int32))
    gi = jnp.stack([ii, ii + N_PAD])
    gj = jnp.stack([jj, jj + N_PAD])
    si = jnp.broadcast_to(ii, (N_CORES, e_pad))
    sj = jnp.broadcast_to(jj, (N_CORES, e_pad))
    idxall = jnp.stack([gi, gj, si, sj], axis=1)
    idxall = idxall.reshape(N_CORES, 4, e_pad // CHUNK, CHUNK)
    idxall = idxall.transpose(0, 2, 1, 3)

    ones_h = jnp.ones((CHUNK,), jnp.float32)
    w16 = jnp.broadcast_to(W.astype(jnp.float32).reshape(()), (16,))

    o = _run(xr, idxall, ones_h, w16, nch)
    out = jnp.concatenate([o[:n], o[N_PAD:N_PAD + n]], axis=1).T[None]
    return out
